# baseline (speedup 1.0000x reference)
import functools

import jax
import jax.numpy as jnp
from jax import lax
from jax.experimental import pallas as pl
from jax.experimental.pallas import tpu as pltpu
from jax.experimental.pallas import tpu_sc as plsc


def _tc1_body(x_ref, win_ref, bin_ref, ww_ref, bw_ref, h_ref, wa_ref):
    x = x_ref[...]
    h = jnp.maximum(
        jnp.dot(x, win_ref[...], preferred_element_type=jnp.float32) + bin_ref[...],
        0.0,
    )
    h_ref[...] = h
    wa_ref[...] = jnp.tanh(
        jnp.dot(h, ww_ref[...], preferred_element_type=jnp.float32) + bw_ref[...]
    )


def _tc1(x, W_in, b_in, W_write, b_write, block_b):
    B, IN = x.shape
    H = W_in.shape[1]
    NQ = W_write.shape[1]
    grid = (B // block_b,)
    return pl.pallas_call(
        _tc1_body,
        grid=grid,
        in_specs=[
            pl.BlockSpec((block_b, IN), lambda i: (i, 0)),
            pl.BlockSpec((IN, H), lambda i: (0, 0)),
            pl.BlockSpec((1, H), lambda i: (0, 0)),
            pl.BlockSpec((H, NQ), lambda i: (0, 0)),
            pl.BlockSpec((1, NQ), lambda i: (0, 0)),
        ],
        out_specs=[
            pl.BlockSpec((block_b, H), lambda i: (i, 0)),
            pl.BlockSpec((block_b, NQ), lambda i: (i, 0)),
        ],
        out_shape=[
            jax.ShapeDtypeStruct((B, H), jnp.float32),
            jax.ShapeDtypeStruct((B, NQ), jnp.float32),
        ],
    )(x, W_in, b_in, W_write, b_write)


_COS_PI_COEF = (
    1.0000000000e+00, -4.9348022005e+00, 4.0587121264e+00, -1.3352627333e+00,
    2.3533063036e-01, -2.5806263764e-02, 1.9285058936e-03, -1.0356905004e-04,
    3.7195286630e-06,
)


def _cos_pi(t):
    u = t * t
    acc = jnp.full_like(u, _COS_PI_COEF[-1])
    for c in _COS_PI_COEF[-2::-1]:
        acc = acc * u + c
    return acc


def _tc2_body(h_ref, ra_ref, wr_ref, br_ref, w1_ref, b1_ref, w2_ref, b2t_ref,
              out_ref):
    H = h_ref.shape[1]
    w1a = w1_ref[:H, :]
    w1b = w1_ref[H:, :]
    wrb = jnp.dot(wr_ref[...], w1b, preferred_element_type=jnp.float32)
    bc = jnp.dot(br_ref[...], w1b, preferred_element_type=jnp.float32) + b1_ref[...]
    e = _cos_pi(ra_ref[...])
    h2 = jnp.maximum(
        jnp.dot(h_ref[...], w1a, preferred_element_type=jnp.float32)
        + jnp.dot(e, wrb, preferred_element_type=jnp.float32)
        + bc,
        0.0,
    )
    out_ref[...] = (
        lax.dot_general(w2_ref[...], h2, (((0,), (1,)), ((), ())),
                        preferred_element_type=jnp.float32)
        + b2t_ref[...]
    )


def _tc2(h, ra, W_read, b_read2, W1, b1, W2, b2t, block_b):
    B, H = h.shape
    NQ = ra.shape[1]
    OUT = W2.shape[1]
    grid = (B // block_b,)
    return pl.pallas_call(
        _tc2_body,
        grid=grid,
        in_specs=[
            pl.BlockSpec((block_b, H), lambda i: (i, 0)),
            pl.BlockSpec((block_b, NQ), lambda i: (i, 0)),
            pl.BlockSpec((NQ, H), lambda i: (0, 0)),
            pl.BlockSpec((1, H), lambda i: (0, 0)),
            pl.BlockSpec((2 * H, H), lambda i: (0, 0)),
            pl.BlockSpec((1, H), lambda i: (0, 0)),
            pl.BlockSpec((H, OUT), lambda i: (0, 0)),
            pl.BlockSpec((OUT, 1), lambda i: (0, 0)),
        ],
        out_specs=pl.BlockSpec((OUT, block_b), lambda i: (0, i)),
        out_shape=jax.ShapeDtypeStruct((OUT, B), jnp.float32),
    )(h, ra, W_read, b_read2, W1, b1, W2, b2t)


_SC_PARAMS = pltpu.CompilerParams(
    needs_layout_passes=False, use_tc_tiling_on_sc=False
)


def _sc_winners(idx, slots):
    B = idx.shape[0]
    info = plsc.get_sparse_core_info()
    NC, NS = 1, info.num_subcores
    chunk = B // (NC * NS)
    mesh = plsc.VectorSubcoreMesh(
        core_axis_name="c", subcore_axis_name="s", num_cores=NC
    )

    @functools.partial(
        pl.kernel,
        mesh=mesh,
        out_type=jax.ShapeDtypeStruct((B,), jnp.int32),
        scratch_types=[
            pltpu.VMEM((B,), jnp.int32),
            pltpu.VMEM((slots,), jnp.int32),
            pltpu.VMEM((chunk,), jnp.int32),
        ],
        compiler_params=_SC_PARAMS,
    )
    def k(idx_hbm, out_hbm, idx_v, owner_v, win_v):
        wid = lax.axis_index("s") * NC + lax.axis_index("c")
        pltpu.sync_copy(idx_hbm, idx_v)

        U = 16

        def scan_body(i, carry):
            for u in range(U):
                v = i * U + u
                iv = idx_v[pl.ds(v * 16, 16)]
                _, last = plsc.scan_count(iv)
                j = lax.iota(jnp.int32, 16) + v * 16
                plsc.store_scatter(owner_v, [iv], j, mask=last)
            return carry

        lax.fori_loop(0, B // 16 // U, scan_body, 0)

        base = wid * chunk

        def win_body(c, carry):
            iv = idx_v[pl.ds(base + c * 16, 16)]
            win_v[pl.ds(c * 16, 16)] = plsc.load_gather(owner_v, [iv])
            return carry

        lax.fori_loop(0, chunk // 16, win_body, 0)
        pltpu.sync_copy(win_v, out_hbm.at[pl.ds(base, chunk)])

    return k(idx)


def _sc_gather_rows(win, wa):
    B, NQ = wa.shape
    info = plsc.get_sparse_core_info()
    NC, NS = 1, info.num_subcores
    chunk = B // (NC * NS)
    n_sub = chunk // 128
    mesh = plsc.VectorSubcoreMesh(
        core_axis_name="c", subcore_axis_name="s", num_cores=NC
    )

    @functools.partial(
        pl.kernel,
        mesh=mesh,
        out_type=jax.ShapeDtypeStruct((B, NQ), jnp.float32),
        scratch_types=[
            pltpu.VMEM((chunk,), jnp.int32),
            pltpu.VMEM((chunk, NQ), jnp.float32),
            pltpu.SemaphoreType.DMA,
        ],
        compiler_params=_SC_PARAMS,
    )
    def k(win_hbm, wa_hbm, out_hbm, win_v, ra_v, sem):
        wid = lax.axis_index("s") * NC + lax.axis_index("c")
        base = wid * chunk
        pltpu.sync_copy(win_hbm.at[pl.ds(base, chunk)], win_v)
        cps = [
            pltpu.async_copy(
                wa_hbm.at[win_v.at[pl.ds(s * 128, 128)]],
                ra_v.at[pl.ds(s * 128, 128)],
                sem,
            )
            for s in range(n_sub)
        ]
        for c in cps:
            c.wait()
        pltpu.sync_copy(ra_v, out_hbm.at[pl.ds(base, chunk)])

    return k(win, wa)


def kernel(x_batch, memory_slot_indices_batch, mem_angles,
           W_in, b_in, W_write, b_write, W_read, b_read,
           W1, b1, W2, b2):
    H = W_in.shape[1]
    NQ = W_write.shape[1]
    OUT = W2.shape[1]
    slots = mem_angles.shape[0]

    win = _sc_winners(memory_slot_indices_batch, slots)
    h, wa = _tc1(x_batch, W_in, b_in.reshape(1, H), W_write,
                 b_write.reshape(1, NQ), block_b=512)
    ra = _sc_gather_rows(win, wa)
    final_t = _tc2(h, ra, W_read, b_read.reshape(1, H), W1,
                   b1.reshape(1, H), W2, b2.reshape(OUT, 1), block_b=512)
    return (final_t.T, h)


# speedup vs baseline: 1.2667x; 1.2667x over previous
import functools

import jax
import jax.numpy as jnp
from jax import lax
from jax.experimental import pallas as pl
from jax.experimental.pallas import tpu as pltpu
from jax.experimental.pallas import tpu_sc as plsc


def _tc1_body(x_ref, win_ref, bin_ref, ww_ref, bw_ref, h_ref, wa_ref):
    x = x_ref[...]
    h = jnp.maximum(
        jnp.dot(x, win_ref[...], preferred_element_type=jnp.float32) + bin_ref[...],
        0.0,
    )
    h_ref[...] = h
    wa_ref[...] = jnp.tanh(
        jnp.dot(h, ww_ref[...], preferred_element_type=jnp.float32) + bw_ref[...]
    )


def _tc1(x, W_in, b_in, W_write, b_write, block_b):
    B, IN = x.shape
    H = W_in.shape[1]
    NQ = W_write.shape[1]
    grid = (B // block_b,)
    return pl.pallas_call(
        _tc1_body,
        grid=grid,
        in_specs=[
            pl.BlockSpec((block_b, IN), lambda i: (i, 0)),
            pl.BlockSpec((IN, H), lambda i: (0, 0)),
            pl.BlockSpec((1, H), lambda i: (0, 0)),
            pl.BlockSpec((H, NQ), lambda i: (0, 0)),
            pl.BlockSpec((1, NQ), lambda i: (0, 0)),
        ],
        out_specs=[
            pl.BlockSpec((block_b, H), lambda i: (i, 0)),
            pl.BlockSpec((block_b, NQ), lambda i: (i, 0)),
        ],
        out_shape=[
            jax.ShapeDtypeStruct((B, H), jnp.float32),
            jax.ShapeDtypeStruct((B, NQ), jnp.float32),
        ],
    )(x, W_in, b_in, W_write, b_write)


_COS_PI_COEF = (
    1.0000000000e+00, -4.9348022005e+00, 4.0587121264e+00, -1.3352627333e+00,
    2.3533063036e-01, -2.5806263764e-02, 1.9285058936e-03, -1.0356905004e-04,
    3.7195286630e-06,
)


def _cos_pi(t):
    u = t * t
    acc = jnp.full_like(u, _COS_PI_COEF[-1])
    for c in _COS_PI_COEF[-2::-1]:
        acc = acc * u + c
    return acc


def _tc2_body(h_ref, ra_ref, wr_ref, br_ref, w1_ref, b1_ref, w2_ref, b2t_ref,
              out_ref):
    H = h_ref.shape[1]
    w1a = w1_ref[:H, :]
    w1b = w1_ref[H:, :]
    wrb = jnp.dot(wr_ref[...], w1b, preferred_element_type=jnp.float32)
    bc = jnp.dot(br_ref[...], w1b, preferred_element_type=jnp.float32) + b1_ref[...]
    e = _cos_pi(ra_ref[...])
    h2 = jnp.maximum(
        jnp.dot(h_ref[...], w1a, preferred_element_type=jnp.float32)
        + jnp.dot(e, wrb, preferred_element_type=jnp.float32)
        + bc,
        0.0,
    )
    out_ref[...] = (
        lax.dot_general(w2_ref[...], h2, (((0,), (1,)), ((), ())),
                        preferred_element_type=jnp.float32)
        + b2t_ref[...]
    )


def _tc2(h, ra, W_read, b_read2, W1, b1, W2, b2t, block_b):
    B, H = h.shape
    NQ = ra.shape[1]
    OUT = W2.shape[1]
    grid = (B // block_b,)
    return pl.pallas_call(
        _tc2_body,
        grid=grid,
        in_specs=[
            pl.BlockSpec((block_b, H), lambda i: (i, 0)),
            pl.BlockSpec((block_b, NQ), lambda i: (i, 0)),
            pl.BlockSpec((NQ, H), lambda i: (0, 0)),
            pl.BlockSpec((1, H), lambda i: (0, 0)),
            pl.BlockSpec((2 * H, H), lambda i: (0, 0)),
            pl.BlockSpec((1, H), lambda i: (0, 0)),
            pl.BlockSpec((H, OUT), lambda i: (0, 0)),
            pl.BlockSpec((OUT, 1), lambda i: (0, 0)),
        ],
        out_specs=pl.BlockSpec((OUT, block_b), lambda i: (0, i)),
        out_shape=jax.ShapeDtypeStruct((OUT, B), jnp.float32),
    )(h, ra, W_read, b_read2, W1, b1, W2, b2t)


_SC_PARAMS = pltpu.CompilerParams(
    needs_layout_passes=False, use_tc_tiling_on_sc=False
)


def _sc_winners(idx, slots):
    B = idx.shape[0]
    info = plsc.get_sparse_core_info()
    NC, NS = 1, info.num_subcores
    chunk = B // (NC * NS)
    mesh = plsc.VectorSubcoreMesh(
        core_axis_name="c", subcore_axis_name="s", num_cores=NC
    )

    @functools.partial(
        pl.kernel,
        mesh=mesh,
        out_type=jax.ShapeDtypeStruct((B,), jnp.int32),
        scratch_types=[
            pltpu.VMEM((B,), jnp.int32),
            pltpu.VMEM((slots,), jnp.int32),
            pltpu.VMEM((chunk,), jnp.int32),
        ],
        compiler_params=_SC_PARAMS,
    )
    def k(idx_hbm, out_hbm, idx_v, owner_v, win_v):
        wid = lax.axis_index("s") * NC + lax.axis_index("c")
        pltpu.sync_copy(idx_hbm, idx_v)

        U = 8

        def scan_body(i, carry):
            for u in range(U):
                v = i * U + u
                iv = idx_v[pl.ds(v * 16, 16)]
                _, last = plsc.scan_count(iv)
                j = lax.iota(jnp.int32, 16) + v * 16
                plsc.store_scatter(owner_v, [iv], j, mask=last)
            return carry

        lax.fori_loop(0, B // 16 // U, scan_body, 0)

        base = wid * chunk

        def win_body(c, carry):
            iv = idx_v[pl.ds(base + c * 16, 16)]
            win_v[pl.ds(c * 16, 16)] = plsc.load_gather(owner_v, [iv])
            return carry

        lax.fori_loop(0, chunk // 16, win_body, 0)
        pltpu.sync_copy(win_v, out_hbm.at[pl.ds(base, chunk)])

    return k(idx)


def _sc_gather_rows(win, wa):
    B, NQ = wa.shape
    info = plsc.get_sparse_core_info()
    NC, NS = 1, info.num_subcores
    chunk = B // (NC * NS)
    n_sub = chunk // 128
    mesh = plsc.VectorSubcoreMesh(
        core_axis_name="c", subcore_axis_name="s", num_cores=NC
    )

    @functools.partial(
        pl.kernel,
        mesh=mesh,
        out_type=jax.ShapeDtypeStruct((B, NQ), jnp.float32),
        scratch_types=[
            pltpu.VMEM((chunk,), jnp.int32),
            pltpu.VMEM((chunk, NQ), jnp.float32),
            pltpu.SemaphoreType.DMA,
        ],
        compiler_params=_SC_PARAMS,
    )
    def k(win_hbm, wa_hbm, out_hbm, win_v, ra_v, sem):
        wid = lax.axis_index("s") * NC + lax.axis_index("c")
        base = wid * chunk
        pltpu.sync_copy(win_hbm.at[pl.ds(base, chunk)], win_v)
        cps = [
            pltpu.async_copy(
                wa_hbm.at[win_v.at[pl.ds(s * 128, 128)]],
                ra_v.at[pl.ds(s * 128, 128)],
                sem,
            )
            for s in range(n_sub)
        ]
        for c in cps:
            c.wait()
        pltpu.sync_copy(ra_v, out_hbm.at[pl.ds(base, chunk)])

    return k(win, wa)


def kernel(x_batch, memory_slot_indices_batch, mem_angles,
           W_in, b_in, W_write, b_write, W_read, b_read,
           W1, b1, W2, b2):
    H = W_in.shape[1]
    NQ = W_write.shape[1]
    OUT = W2.shape[1]
    slots = mem_angles.shape[0]

    win = _sc_winners(memory_slot_indices_batch, slots)
    h, wa = _tc1(x_batch, W_in, b_in.reshape(1, H), W_write,
                 b_write.reshape(1, NQ), block_b=1024)
    ra = _sc_gather_rows(win, wa)
    final_t = _tc2(h, ra, W_read, b_read.reshape(1, H), W1,
                   b1.reshape(1, H), W2, b2.reshape(OUT, 1), block_b=1024)
    return (final_t.T, h)
